# baseline (speedup 1.0000x reference)
import jax
import jax.numpy as jnp
from jax import lax
from jax.experimental import pallas as pl
from jax.experimental.pallas import tpu as pltpu, tpu_sc as plsc

BATCH = 4096
NCOL = 16384
GRID_W = 128

NC, NS, NL = 2, 16, 16
NW = NC * NS
RW = BATCH // NW
NVREG = NCOL // NL
UNROLL = 4
STEPS = NVREG // UNROLL
GROUP = NL // 2
NGROUP = RW // GROUP


def _merge(v1, i1, v2, i2):
    take2 = (v2 < v1) | ((v2 == v1) & (i2 < i1))
    return jnp.where(take2, v2, v1), jnp.where(take2, i2, i1)


def _row_argmin(row_ref):
    inf = jnp.full((NL,), jnp.inf, jnp.float32)
    zero = jnp.zeros((NL,), jnp.int32)
    init = (inf,) * UNROLL + (zero,) * UNROLL

    def body(i, carry):
        bv = list(carry[:UNROLL])
        bj = list(carry[UNROLL:])
        for u in range(UNROLL):
            jj = i * UNROLL + u
            v = row_ref[pl.ds(jj * NL, NL)]
            m = v < bv[u]
            bv[u] = jnp.where(m, v, bv[u])
            bj[u] = jnp.where(m, jnp.full((NL,), jj, jnp.int32), bj[u])
        return tuple(bv) + tuple(bj)

    res = lax.fori_loop(0, STEPS, body, init)
    lane = lax.iota(jnp.int32, NL)
    bv, bi = res[0], res[UNROLL] * NL + lane
    for u in range(1, UNROLL):
        bv, bi = _merge(bv, bi, res[u], res[UNROLL + u] * NL + lane)
    minv = jnp.full((NL,), jnp.sort(bv)[0], jnp.float32)
    cand = jnp.where(bv == minv, bi, jnp.full((NL,), NCOL, jnp.int32))
    return jnp.sort(cand)[0]


def _body(d_hbm, out_hbm, buf0, buf1, outb, sem0, sem1):
    wid = lax.axis_index("s") * NC + lax.axis_index("c")
    base = wid * RW
    bufs = (buf0, buf1)
    sems = (sem0, sem1)
    lane = lax.iota(jnp.int32, NL)

    pltpu.async_copy(d_hbm.at[base], buf0, sem0)
    pltpu.async_copy(d_hbm.at[base + 1], buf1, sem1)

    def group(g, carry):
        res = jnp.zeros((NL,), jnp.float32)
        for q in range(GROUP):
            s = q % 2
            r = g * GROUP + q
            row = base + r
            pltpu.make_async_copy(d_hbm.at[row], bufs[s], sems[s]).wait()
            midx = _row_argmin(bufs[s])
            rowf = midx.astype(jnp.float32) * (1.0 / GRID_W)
            colf = (midx & (GRID_W - 1)).astype(jnp.float32)
            res = jnp.where(lane == 2 * q, rowf,
                            jnp.where(lane == 2 * q + 1, colf, res))

            @pl.when(r + 2 < RW)
            def _():
                pltpu.async_copy(d_hbm.at[row + 2], bufs[s], sems[s])
        outb[pl.ds(g * NL, NL)] = res
        return carry

    lax.fori_loop(0, NGROUP, group, 0)
    pltpu.sync_copy(outb, out_hbm.at[wid])


@jax.jit
def kernel(distances):
    mesh = plsc.VectorSubcoreMesh(core_axis_name="c", subcore_axis_name="s")
    f = pl.kernel(
        _body,
        out_type=jax.ShapeDtypeStruct((NW, RW * 2), jnp.float32),
        mesh=mesh,
        compiler_params=pltpu.CompilerParams(needs_layout_passes=False),
        scratch_types=[
            pltpu.VMEM((NCOL,), jnp.float32),
            pltpu.VMEM((NCOL,), jnp.float32),
            pltpu.VMEM((RW * 2,), jnp.float32),
            pltpu.SemaphoreType.DMA,
            pltpu.SemaphoreType.DMA,
        ],
    )
    return f(distances).reshape(BATCH, 2)


# speedup vs baseline: 1.0925x; 1.0925x over previous
import jax
import jax.numpy as jnp
from jax import lax
from jax.experimental import pallas as pl
from jax.experimental.pallas import tpu as pltpu, tpu_sc as plsc

BATCH = 4096
NCOL = 16384
GRID_W = 128

NC, NS, NL = 2, 16, 16
NW = NC * NS
RW = BATCH // NW
NVREG = NCOL // NL
UNROLL = 8
STEPS = NVREG // UNROLL
GROUP = NL // 2
NGROUP = RW // GROUP


def _merge(v1, i1, v2, i2):
    take2 = (v2 < v1) | ((v2 == v1) & (i2 < i1))
    return jnp.where(take2, v2, v1), jnp.where(take2, i2, i1)


def _row_argmin(row_ref):
    inf = jnp.full((NL,), jnp.inf, jnp.float32)
    zero = jnp.zeros((NL,), jnp.int32)
    init = (inf,) * UNROLL + (zero,) * UNROLL

    def body(i, carry):
        bv = list(carry[:UNROLL])
        bj = list(carry[UNROLL:])
        for u in range(UNROLL):
            jj = i * UNROLL + u
            v = row_ref[pl.ds(jj * NL, NL)]
            m = v < bv[u]
            bv[u] = jnp.where(m, v, bv[u])
            bj[u] = jnp.where(m, jnp.full((NL,), jj, jnp.int32), bj[u])
        return tuple(bv) + tuple(bj)

    res = lax.fori_loop(0, STEPS, body, init)
    lane = lax.iota(jnp.int32, NL)
    bv, bi = res[0], res[UNROLL] * NL + lane
    for u in range(1, UNROLL):
        bv, bi = _merge(bv, bi, res[u], res[UNROLL + u] * NL + lane)
    minv = jnp.full((NL,), jnp.sort(bv)[0], jnp.float32)
    cand = jnp.where(bv == minv, bi, jnp.full((NL,), NCOL, jnp.int32))
    return jnp.sort(cand)[0]


def _body(d_hbm, out_hbm, buf0, buf1, outb, sem0, sem1):
    wid = lax.axis_index("s") * NC + lax.axis_index("c")
    base = wid * RW
    bufs = (buf0, buf1)
    sems = (sem0, sem1)
    lane = lax.iota(jnp.int32, NL)

    pltpu.async_copy(d_hbm.at[base], buf0, sem0)
    pltpu.async_copy(d_hbm.at[base + 1], buf1, sem1)

    def group(g, carry):
        res = jnp.zeros((NL,), jnp.float32)
        for q in range(GROUP):
            s = q % 2
            r = g * GROUP + q
            row = base + r
            pltpu.make_async_copy(d_hbm.at[row], bufs[s], sems[s]).wait()
            midx = _row_argmin(bufs[s])
            rowf = midx.astype(jnp.float32) * (1.0 / GRID_W)
            colf = (midx & (GRID_W - 1)).astype(jnp.float32)
            res = jnp.where(lane == 2 * q, rowf,
                            jnp.where(lane == 2 * q + 1, colf, res))

            @pl.when(r + 2 < RW)
            def _():
                pltpu.async_copy(d_hbm.at[row + 2], bufs[s], sems[s])
        outb[pl.ds(g * NL, NL)] = res
        return carry

    lax.fori_loop(0, NGROUP, group, 0)
    pltpu.sync_copy(outb, out_hbm.at[wid])


@jax.jit
def kernel(distances):
    mesh = plsc.VectorSubcoreMesh(core_axis_name="c", subcore_axis_name="s")
    f = pl.kernel(
        _body,
        out_type=jax.ShapeDtypeStruct((NW, RW * 2), jnp.float32),
        mesh=mesh,
        compiler_params=pltpu.CompilerParams(needs_layout_passes=False),
        scratch_types=[
            pltpu.VMEM((NCOL,), jnp.float32),
            pltpu.VMEM((NCOL,), jnp.float32),
            pltpu.VMEM((RW * 2,), jnp.float32),
            pltpu.SemaphoreType.DMA,
            pltpu.SemaphoreType.DMA,
        ],
    )
    return f(distances).reshape(BATCH, 2)


# speedup vs baseline: 1.5215x; 1.3927x over previous
import jax
import jax.numpy as jnp
from jax import lax
from jax.experimental import pallas as pl
from jax.experimental.pallas import tpu as pltpu, tpu_sc as plsc

BATCH = 4096
NCOL = 16384
GRID_W = 128

NC, NS, NL = 2, 16, 16
NW = NC * NS
RW = BATCH // NW
NVREG = NCOL // NL
UNROLL = 8
STEPS = NVREG // UNROLL
GROUP = NL // 2
NGROUP = RW // GROUP


def _merge(v1, i1, v2, i2):
    take2 = (v2 < v1) | ((v2 == v1) & (i2 < i1))
    return jnp.where(take2, v2, v1), jnp.where(take2, i2, i1)


def _row_argmin(row_ref):
    inf = jnp.full((NL,), jnp.inf, jnp.float32)
    zero = jnp.zeros((NL,), jnp.int32)
    init = (inf,) * UNROLL + (zero,) * UNROLL

    def body(i, carry):
        bv = list(carry[:UNROLL])
        bj = list(carry[UNROLL:])
        for u in range(UNROLL):
            jj = i * UNROLL + u
            v = row_ref[pl.ds(jj * NL, NL)]
            m = v < bv[u]
            bv[u] = jnp.where(m, v, bv[u])
            bj[u] = jnp.where(m, jnp.full((NL,), jj, jnp.int32), bj[u])
        return tuple(bv) + tuple(bj)

    res = lax.fori_loop(0, STEPS, body, init)
    lane = lax.iota(jnp.int32, NL)
    bv, bi = res[0], res[UNROLL] * NL + lane
    for u in range(1, UNROLL):
        bv, bi = _merge(bv, bi, res[u], res[UNROLL + u] * NL + lane)
    minv = jnp.full((NL,), jnp.sort(bv)[0], jnp.float32)
    cand = jnp.where(bv == minv, bi, jnp.full((NL,), NCOL, jnp.int32))
    return jnp.sort(cand)[0]


NBUF = 4


def _body(d_hbm, out_hbm, buf0, buf1, buf2, buf3, outb,
          sem0, sem1, sem2, sem3):
    wid = lax.axis_index("s") * NC + lax.axis_index("c")
    base = wid * RW
    bufs = (buf0, buf1, buf2, buf3)
    sems = (sem0, sem1, sem2, sem3)
    lane = lax.iota(jnp.int32, NL)

    for s in range(NBUF - 1):
        pltpu.async_copy(d_hbm.at[base + s], bufs[s], sems[s])

    def group(g, carry):
        res = jnp.zeros((NL,), jnp.float32)
        for q in range(GROUP):
            s = q % NBUF
            sn = (q + NBUF - 1) % NBUF
            r = g * GROUP + q
            row = base + r
            pltpu.make_async_copy(d_hbm.at[row], bufs[s], sems[s]).wait()

            @pl.when(r + NBUF - 1 < RW)
            def _():
                pltpu.async_copy(
                    d_hbm.at[row + NBUF - 1], bufs[sn], sems[sn])

            midx = _row_argmin(bufs[s])
            rowf = midx.astype(jnp.float32) * (1.0 / GRID_W)
            colf = (midx & (GRID_W - 1)).astype(jnp.float32)
            res = jnp.where(lane == 2 * q, rowf,
                            jnp.where(lane == 2 * q + 1, colf, res))
        outb[pl.ds(g * NL, NL)] = res
        return carry

    lax.fori_loop(0, NGROUP, group, 0)
    pltpu.sync_copy(outb, out_hbm.at[wid])


@jax.jit
def kernel(distances):
    mesh = plsc.VectorSubcoreMesh(core_axis_name="c", subcore_axis_name="s")
    f = pl.kernel(
        _body,
        out_type=jax.ShapeDtypeStruct((NW, RW * 2), jnp.float32),
        mesh=mesh,
        compiler_params=pltpu.CompilerParams(needs_layout_passes=False),
        scratch_types=(
            [pltpu.VMEM((NCOL,), jnp.float32)] * NBUF
            + [pltpu.VMEM((RW * 2,), jnp.float32)]
            + [pltpu.SemaphoreType.DMA] * NBUF
        ),
    )
    return f(distances).reshape(BATCH, 2)
